# Initial kernel scaffold; baseline (speedup 1.0000x reference)
import jax
import jax.numpy as jnp
from jax.experimental import pallas as pl


def kernel(n0, n1, n2, n3, n4, n5, n6, n7, n8, n9, n10, n11, n12, n13, n14, n15, w_fine, w_medium, w_coarse):
    raise NotImplementedError("write your pallas kernel here")


# speedup vs baseline: 7.7601x; 7.7601x over previous
import functools

import jax
import jax.numpy as jnp
from jax.experimental import pallas as pl
from jax.experimental.pallas import tpu as pltpu

N_NODES = 16
BATCH = 8192
DIM = 512
TB = 256

_SPACINGS = (2, 3, 5)


def _neighbors(j):
    result = []
    for s_idx, sp in enumerate(_SPACINGS):
        for i in (j - sp, j + sp):
            if 0 <= i < N_NODES:
                result.append((i, s_idx))
    return result


def _body(w_ref, *refs):
    in_refs = refs[:N_NODES]
    out_ref = refs[N_NODES]
    w = [w_ref[0], w_ref[1], w_ref[2]]
    for j in range(N_NODES):
        acc = None
        for s_idx in range(3):
            terms = [in_refs[i][...] for (i, si) in _neighbors(j) if si == s_idx]
            if not terms:
                continue
            t = terms[0]
            for extra in terms[1:]:
                t = t + extra
            t = t * w[s_idx]
            acc = t if acc is None else acc + t
        out_ref[j] = acc


def kernel(n0, n1, n2, n3, n4, n5, n6, n7, n8, n9, n10, n11, n12, n13, n14,
           n15, w_fine, w_medium, w_coarse):
    nodes = [n0, n1, n2, n3, n4, n5, n6, n7, n8, n9, n10, n11, n12, n13, n14, n15]
    w = jnp.stack([w_fine, w_medium, w_coarse])
    grid = (BATCH // TB,)
    in_specs = [pl.BlockSpec(memory_space=pltpu.SMEM)] + [
        pl.BlockSpec((TB, DIM), lambda i: (i, 0)) for _ in range(N_NODES)
    ]
    out_specs = pl.BlockSpec((N_NODES, TB, DIM), lambda i: (0, i, 0))
    return pl.pallas_call(
        _body,
        grid=grid,
        in_specs=in_specs,
        out_specs=out_specs,
        out_shape=jax.ShapeDtypeStruct((N_NODES, BATCH, DIM), jnp.float32),
        compiler_params=pltpu.CompilerParams(
            dimension_semantics=("parallel",),
        ),
    )(w, *nodes)
